# baseline (speedup 1.0000x reference)
import functools

import jax
import jax.numpy as jnp
from jax import lax
from jax.experimental import pallas as pl
from jax.experimental.pallas import tpu as pltpu
from jax.experimental.pallas import tpu_sc as plsc

K = 16
C = 3
B_, N_ = 8, 4096
RB = 128
CHUNK = 128
NCH = N_ // CHUNK
MAXI = 0x7FFFFFFF


def _topk_body(vall_ref, vrow_ref, idx_ref):
    va = vall_ref[0]
    vb = vrow_ref[0]
    g = lax.dot_general(va, vb, (((1,), (1,)), ((), ())),
                        preferred_element_type=jnp.float32)
    sqa = jnp.sum(va * va, axis=1, keepdims=True)
    sqb = jnp.sum(vb * vb, axis=1)[None, :]
    dist = sqa - 2.0 * g + sqb
    bits = lax.bitcast_convert_type(jnp.maximum(dist, 0.0), jnp.int32)
    b3 = bits.reshape(NCH, CHUNK, RB)
    li3 = lax.broadcasted_iota(jnp.int32, (NCH, CHUNK, RB), 1)
    p = (b3 & jnp.int32(-CHUNK)) | li3
    fi = lax.broadcasted_iota(jnp.int32, (NCH, CHUNK, RB), 0) * CHUNK + li3
    ci = lax.broadcasted_iota(jnp.int32, (NCH, RB), 0)
    ti = lax.broadcasted_iota(jnp.int32, (K, RB), 0)
    sri = lax.broadcasted_iota(jnp.int32, (K, NCH, RB), 0)
    self_idx = pl.program_id(1) * RB + lax.broadcasted_iota(
        jnp.int32, (1, RB), 1)
    p = jnp.where(fi == self_idx.reshape(1, 1, RB), jnp.int32(MAXI), p)

    def round_cond(carry):
        _, _, _, r, done = carry
        return jnp.logical_and(r < K, done == 0)

    def round_body(carry):
        p, slab, gslab, r, _ = carry
        m2 = jnp.min(p, axis=1)
        m_rem = jnp.min(m2, axis=0, keepdims=True)
        cnt = jnp.sum((slab < m_rem[None]).astype(jnp.int32),
                      axis=(0, 1))
        done = (jnp.min(cnt) >= K).astype(jnp.int32)
        lidx2 = m2 & (CHUNK - 1)
        g2 = ci * CHUNK + lidx2
        p = jnp.where(li3 == lidx2[:, None, :], jnp.int32(MAXI), p)
        slab = jnp.where(sri == r, m2[None], slab)
        gslab = jnp.where(sri == r, g2[None], gslab)
        return (p, slab, gslab, r + 1, done)

    slab0 = jnp.full((K, NCH, RB), MAXI, jnp.int32)
    _, slab, gslab, _, _ = lax.while_loop(
        round_cond, round_body,
        (p, slab0, slab0, jnp.int32(0), jnp.int32(0)))

    def ext(t, carry):
        slab, acc = carry
        m = jnp.min(slab, axis=(0, 1), keepdims=True)
        msk = slab == m
        gidx = jnp.min(jnp.where(msk, gslab, jnp.int32(MAXI)),
                       axis=(0, 1))
        slab = jnp.where(msk, jnp.int32(MAXI), slab)
        acc = jnp.where(ti == t, gidx[None, :], acc)
        return (slab, acc)

    _, acc = lax.fori_loop(0, K, ext,
                           (slab, jnp.zeros((K, RB), jnp.int32)))
    idx_ref[0] = acc


def _tc_topk(vertices):
    b, n, _ = vertices.shape
    return pl.pallas_call(
        _topk_body,
        grid=(b, n // RB),
        in_specs=[
            pl.BlockSpec((1, n, C), lambda i, j: (i, 0, 0)),
            pl.BlockSpec((1, RB, C), lambda i, j: (i, j, 0)),
        ],
        out_specs=pl.BlockSpec((1, K, RB), lambda i, j: (i, 0, j)),
        out_shape=jax.ShapeDtypeStruct((b, K, n), jnp.int32),
    )(vertices, vertices)


ROWS_PER_W = N_ * B_ // 32
SUB = 256
NSUB = ROWS_PER_W // SUB


def _sc_gather(v_flat, idx_t):
    mesh = plsc.VectorSubcoreMesh(core_axis_name="c", subcore_axis_name="s")

    @functools.partial(
        pl.kernel,
        mesh=mesh,
        out_type=jax.ShapeDtypeStruct((B_ * N_ * K * 2 * C,), jnp.float32),
        compiler_params=pltpu.CompilerParams(needs_layout_passes=False),
        scratch_types=[
            pltpu.VMEM((N_ * C,), jnp.float32),
            pltpu.VMEM((K, SUB), jnp.int32),
            pltpu.VMEM((SUB * K * 2 * C,), jnp.float32),
        ],
    )
    def body(v_hbm, idx_hbm, out_hbm, vf, idxb, outb):
        wid = lax.axis_index("c") * 16 + lax.axis_index("s")
        b = wid // 4
        q = wid % 4
        pltpu.sync_copy(v_hbm.at[pl.ds(b * N_ * C, N_ * C)], vf)
        i6 = lax.iota(jnp.int32, 16) * jnp.int32(2 * C)
        t_iota = lax.iota(jnp.int32, 16)

        for s in range(NSUB):
            row0 = q * ROWS_PER_W + s * SUB
            pltpu.sync_copy(idx_hbm.at[b, pl.ds(0, K), pl.ds(row0, SUB)],
                            idxb)

            def rb(r, carry):
                rv = jnp.broadcast_to(r, (16,)).astype(jnp.int32)
                iv3 = plsc.load_gather(idxb, [t_iota, rv]) * 3
                civ3 = jnp.broadcast_to((row0 + r) * 3, (16,)).astype(
                    jnp.int32)
                nx = plsc.load_gather(vf, [iv3])
                ny = plsc.load_gather(vf, [iv3 + 1])
                nz = plsc.load_gather(vf, [iv3 + 2])
                cx = plsc.load_gather(vf, [civ3])
                cy = plsc.load_gather(vf, [civ3 + 1])
                cz = plsc.load_gather(vf, [civ3 + 2])
                off = r * jnp.int32(K * 2 * C) + i6
                plsc.store_scatter(outb, [off + 0], nx - cx)
                plsc.store_scatter(outb, [off + 1], ny - cy)
                plsc.store_scatter(outb, [off + 2], nz - cz)
                plsc.store_scatter(outb, [off + 3], cx)
                plsc.store_scatter(outb, [off + 4], cy)
                plsc.store_scatter(outb, [off + 5], cz)
                return carry

            lax.fori_loop(0, SUB, rb, 0)
            goff = (b * N_ + row0) * K * 2 * C
            pltpu.sync_copy(outb, out_hbm.at[pl.ds(goff, SUB * K * 2 * C)])

    return body(v_flat, idx_t)


def kernel(vertices):
    b, n, c = vertices.shape
    idx_t = _tc_topk(vertices)
    out_flat = _sc_gather(vertices.reshape(-1), idx_t)
    return out_flat.reshape(b, n, K, 2 * c)


# speedup vs baseline: 2.1492x; 2.1492x over previous
import functools

import jax
import jax.numpy as jnp
from jax import lax
from jax.experimental import pallas as pl
from jax.experimental.pallas import tpu as pltpu
from jax.experimental.pallas import tpu_sc as plsc

K = 16
C = 3
B_, N_ = 8, 4096
RB = 256
CHUNK = 128
NCH = N_ // CHUNK
MAXI = 0x7FFFFFFF


def _topk_body(vall_ref, vrow_ref, idx_ref):
    va = vall_ref[0]
    vb = vrow_ref[0]
    g = lax.dot_general(va, vb, (((1,), (1,)), ((), ())),
                        preferred_element_type=jnp.float32)
    sqa = jnp.sum(va * va, axis=1, keepdims=True)
    sqb = jnp.sum(vb * vb, axis=1)[None, :]
    dist = sqa - 2.0 * g + sqb
    bits = lax.bitcast_convert_type(jnp.maximum(dist, 0.0), jnp.int32)
    b3 = bits.reshape(NCH, CHUNK, RB)
    li3 = lax.broadcasted_iota(jnp.int32, (NCH, CHUNK, RB), 1)
    p = (b3 & jnp.int32(-CHUNK)) | li3
    fi = lax.broadcasted_iota(jnp.int32, (NCH, CHUNK, RB), 0) * CHUNK + li3
    ci = lax.broadcasted_iota(jnp.int32, (NCH, RB), 0)
    ti = lax.broadcasted_iota(jnp.int32, (K, RB), 0)
    sri = lax.broadcasted_iota(jnp.int32, (K, NCH, RB), 0)
    self_idx = pl.program_id(1) * RB + lax.broadcasted_iota(
        jnp.int32, (1, RB), 1)
    p = jnp.where(fi == self_idx.reshape(1, 1, RB), jnp.int32(MAXI), p)

    def it(t, carry):
        p, acc = carry
        m2 = jnp.min(p, axis=1)
        m = jnp.min(m2, axis=0, keepdims=True)
        cstar = jnp.min(jnp.where(m2 == m, ci, jnp.int32(MAXI)),
                        axis=0, keepdims=True)
        gidx = cstar * CHUNK + (m & (CHUNK - 1))
        p = jnp.where(fi == gidx.reshape(1, 1, RB), jnp.int32(MAXI), p)
        acc = jnp.where(ti == t, gidx, acc)
        return (p, acc)

    _, acc = lax.fori_loop(0, K, it, (p, jnp.zeros((K, RB), jnp.int32)))
    idx_ref[0] = acc


def _tc_topk(vertices):
    b, n, _ = vertices.shape
    return pl.pallas_call(
        _topk_body,
        grid=(b, n // RB),
        in_specs=[
            pl.BlockSpec((1, n, C), lambda i, j: (i, 0, 0)),
            pl.BlockSpec((1, RB, C), lambda i, j: (i, j, 0)),
        ],
        out_specs=pl.BlockSpec((1, K, RB), lambda i, j: (i, 0, j)),
        out_shape=jax.ShapeDtypeStruct((b, K, n), jnp.int32),
    )(vertices, vertices)


ROWS_PER_W = N_ * B_ // 32
SUB = 256
NSUB = ROWS_PER_W // SUB


def _sc_gather(v_flat, idx_t):
    mesh = plsc.VectorSubcoreMesh(core_axis_name="c", subcore_axis_name="s")

    @functools.partial(
        pl.kernel,
        mesh=mesh,
        out_type=jax.ShapeDtypeStruct((B_ * N_ * K * 2 * C,), jnp.float32),
        compiler_params=pltpu.CompilerParams(needs_layout_passes=False),
        scratch_types=[
            pltpu.VMEM((N_ * C,), jnp.float32),
            pltpu.VMEM((K, SUB), jnp.int32),
            pltpu.VMEM((SUB * K * 2 * C,), jnp.float32),
        ],
    )
    def body(v_hbm, idx_hbm, out_hbm, vf, idxb, outb):
        wid = lax.axis_index("c") * 16 + lax.axis_index("s")
        b = wid // 4
        q = wid % 4
        pltpu.sync_copy(v_hbm.at[pl.ds(b * N_ * C, N_ * C)], vf)
        i6 = lax.iota(jnp.int32, 16) * jnp.int32(2 * C)
        t_iota = lax.iota(jnp.int32, 16)

        for s in range(NSUB):
            row0 = q * ROWS_PER_W + s * SUB
            pltpu.sync_copy(idx_hbm.at[b, pl.ds(0, K), pl.ds(row0, SUB)],
                            idxb)

            def rb(r, carry):
                rv = jnp.broadcast_to(r, (16,)).astype(jnp.int32)
                iv3 = plsc.load_gather(idxb, [t_iota, rv]) * 3
                civ3 = jnp.broadcast_to((row0 + r) * 3, (16,)).astype(
                    jnp.int32)
                nx = plsc.load_gather(vf, [iv3])
                ny = plsc.load_gather(vf, [iv3 + 1])
                nz = plsc.load_gather(vf, [iv3 + 2])
                cx = plsc.load_gather(vf, [civ3])
                cy = plsc.load_gather(vf, [civ3 + 1])
                cz = plsc.load_gather(vf, [civ3 + 2])
                off = r * jnp.int32(K * 2 * C) + i6
                plsc.store_scatter(outb, [off + 0], nx - cx)
                plsc.store_scatter(outb, [off + 1], ny - cy)
                plsc.store_scatter(outb, [off + 2], nz - cz)
                plsc.store_scatter(outb, [off + 3], cx)
                plsc.store_scatter(outb, [off + 4], cy)
                plsc.store_scatter(outb, [off + 5], cz)
                return carry

            lax.fori_loop(0, SUB, rb, 0)
            goff = (b * N_ + row0) * K * 2 * C
            pltpu.sync_copy(outb, out_hbm.at[pl.ds(goff, SUB * K * 2 * C)])

    return body(v_flat, idx_t)


def kernel(vertices):
    b, n, c = vertices.shape
    idx_t = _tc_topk(vertices)
    out_flat = _sc_gather(vertices.reshape(-1), idx_t)
    return out_flat.reshape(b, n, K, 2 * c)


# speedup vs baseline: 2.1784x; 1.0136x over previous
import functools

import jax
import jax.numpy as jnp
from jax import lax
from jax.experimental import pallas as pl
from jax.experimental.pallas import tpu as pltpu
from jax.experimental.pallas import tpu_sc as plsc

K = 16
C = 3
B_, N_ = 8, 4096
RB = 256
CHUNK = 128
NCH = N_ // CHUNK
MAXI = 0x7FFFFFFF


def _topk_body(vall_ref, vrow_ref, idx_ref):
    va = vall_ref[0]
    vb = vrow_ref[0]
    g = lax.dot_general(va, vb, (((1,), (1,)), ((), ())),
                        preferred_element_type=jnp.float32)
    sqa = jnp.sum(va * va, axis=1, keepdims=True)
    sqb = jnp.sum(vb * vb, axis=1)[None, :]
    dist = sqa - 2.0 * g + sqb
    bits = lax.bitcast_convert_type(jnp.maximum(dist, 0.0), jnp.int32)
    b3 = bits.reshape(NCH, CHUNK, RB)
    li3 = lax.broadcasted_iota(jnp.int32, (NCH, CHUNK, RB), 1)
    p = (b3 & jnp.int32(-CHUNK)) | li3
    fi = lax.broadcasted_iota(jnp.int32, (NCH, CHUNK, RB), 0) * CHUNK + li3
    ci = lax.broadcasted_iota(jnp.int32, (NCH, RB), 0)
    ti = lax.broadcasted_iota(jnp.int32, (K, RB), 0)
    sri = lax.broadcasted_iota(jnp.int32, (K, NCH, RB), 0)
    self_idx = pl.program_id(1) * RB + lax.broadcasted_iota(
        jnp.int32, (1, RB), 1)
    p = jnp.where(fi == self_idx.reshape(1, 1, RB), jnp.int32(MAXI), p)

    def it(t, carry):
        p, acc = carry
        m2 = jnp.min(p, axis=1)
        m = jnp.min(m2, axis=0, keepdims=True)
        cstar = jnp.min(jnp.where(m2 == m, ci, jnp.int32(MAXI)),
                        axis=0, keepdims=True)
        gidx = cstar * CHUNK + (m & (CHUNK - 1))
        p = jnp.where(fi == gidx.reshape(1, 1, RB), jnp.int32(MAXI), p)
        acc = jnp.where(ti == t, gidx, acc)
        return (p, acc)

    _, acc = lax.fori_loop(0, K, it, (p, jnp.zeros((K, RB), jnp.int32)))
    idx_ref[0] = acc


def _tc_topk(vertices):
    b, n, _ = vertices.shape
    return pl.pallas_call(
        _topk_body,
        grid=(b, n // RB),
        in_specs=[
            pl.BlockSpec((1, n, C), lambda i, j: (i, 0, 0)),
            pl.BlockSpec((1, RB, C), lambda i, j: (i, j, 0)),
        ],
        out_specs=pl.BlockSpec((1, K, RB), lambda i, j: (i, 0, j)),
        out_shape=jax.ShapeDtypeStruct((b, K, n), jnp.int32),
    )(vertices, vertices)


BH = B_ // 2
ROWS_PER_W = N_ * BH // 32
SUB = 256
NSUB = ROWS_PER_W // SUB


def _sc_gather(v_flat, idx_t):
    mesh = plsc.VectorSubcoreMesh(core_axis_name="c", subcore_axis_name="s")

    @functools.partial(
        pl.kernel,
        mesh=mesh,
        out_type=jax.ShapeDtypeStruct((BH * N_ * K * 2 * C,), jnp.float32),
        compiler_params=pltpu.CompilerParams(needs_layout_passes=False),
        scratch_types=[
            pltpu.VMEM((N_ * C,), jnp.float32),
            pltpu.VMEM((K, SUB), jnp.int32),
            pltpu.VMEM((SUB * K * 2 * C,), jnp.float32),
        ],
    )
    def body(v_hbm, idx_hbm, out_hbm, vf, idxb, outb):
        wid = lax.axis_index("c") * 16 + lax.axis_index("s")
        b = wid // 8
        q = wid % 8
        pltpu.sync_copy(v_hbm.at[pl.ds(b * N_ * C, N_ * C)], vf)
        i6 = lax.iota(jnp.int32, 16) * jnp.int32(2 * C)
        t_iota = lax.iota(jnp.int32, 16)

        for s in range(NSUB):
            row0 = q * ROWS_PER_W + s * SUB
            pltpu.sync_copy(idx_hbm.at[b, pl.ds(0, K), pl.ds(row0, SUB)],
                            idxb)

            def rb(r, carry):
                rv = jnp.broadcast_to(r, (16,)).astype(jnp.int32)
                iv3 = plsc.load_gather(idxb, [t_iota, rv]) * 3
                civ3 = jnp.broadcast_to((row0 + r) * 3, (16,)).astype(
                    jnp.int32)
                nx = plsc.load_gather(vf, [iv3])
                ny = plsc.load_gather(vf, [iv3 + 1])
                nz = plsc.load_gather(vf, [iv3 + 2])
                cx = plsc.load_gather(vf, [civ3])
                cy = plsc.load_gather(vf, [civ3 + 1])
                cz = plsc.load_gather(vf, [civ3 + 2])
                off = r * jnp.int32(K * 2 * C) + i6
                plsc.store_scatter(outb, [off + 0], nx - cx)
                plsc.store_scatter(outb, [off + 1], ny - cy)
                plsc.store_scatter(outb, [off + 2], nz - cz)
                plsc.store_scatter(outb, [off + 3], cx)
                plsc.store_scatter(outb, [off + 4], cy)
                plsc.store_scatter(outb, [off + 5], cz)
                return carry

            lax.fori_loop(0, SUB, rb, 0)
            goff = (b * N_ + row0) * K * 2 * C
            pltpu.sync_copy(outb, out_hbm.at[pl.ds(goff, SUB * K * 2 * C)])

    return body(v_flat, idx_t)


def kernel(vertices):
    b, n, c = vertices.shape
    v1, v2 = vertices[:BH], vertices[BH:]
    idx1 = _tc_topk(v1)
    out1 = _sc_gather(v1.reshape(-1), idx1)
    idx2 = _tc_topk(v2)
    out2 = _sc_gather(v2.reshape(-1), idx2)
    out = jnp.concatenate([out1, out2])
    return out.reshape(b, n, K, 2 * c)
